# baseline (speedup 1.0000x reference)
import functools

import jax
import jax.numpy as jnp
from jax import lax
from jax.experimental import pallas as pl
from jax.experimental.pallas import tpu as pltpu
from jax.experimental.pallas import tpu_sc as plsc

_NBIN = 256
_N = 4096 * 4096
_NWORKERS = 32
_EPW = _N // _NWORKERS
_CHUNK = 16384
_NCHUNK = _EPW // _CHUNK
_LANES = 16
_HROW = 264
_HSIZE = _LANES * _HROW
_NBANK = 4
_HTOT = _NBANK * _HSIZE


def _min_body(x_ref, o_ref):
    i = pl.program_id(0)
    m = jnp.min(x_ref[...])

    @pl.when(i == 0)
    def _():
        o_ref[0, 0] = m

    @pl.when(i > 0)
    def _():
        o_ref[0, 0] = jnp.minimum(o_ref[0, 0], m)


def _global_min(img):
    return pl.pallas_call(
        _min_body,
        grid=(16,),
        in_specs=[pl.BlockSpec((256, 4096), lambda i: (i, 0))],
        out_specs=pl.BlockSpec(memory_space=pltpu.SMEM),
        out_shape=jax.ShapeDtypeStruct((1, 1), jnp.float32),
    )(img)


def _hist_worker(img_hbm, out_hbm, buf, hist, hminv, invdh, wid):
    lane_base = jnp.arange(_LANES, dtype=jnp.int32) * _HROW
    zeros16 = jnp.zeros((_LANES,), jnp.float32)

    def zero_body(j, _):
        hist[pl.ds(j * _LANES, _LANES)] = zeros16
        return _

    lax.fori_loop(0, _HTOT // _LANES, zero_body, None)

    base = wid * _EPW

    def chunk_body(c, _):
        pltpu.sync_copy(img_hbm.at[pl.ds(base + c * _CHUNK, _CHUNK)], buf)

        def vec_body(v, _):
            for s in range(_NBANK):
                x = buf[pl.ds((v * _NBANK + s) * _LANES, _LANES)]
                keep = jnp.logical_and(x >= hminv, x <= 0.0)
                t = (x - hminv) * invdh
                tc = jnp.clip(t, 0.0, 255.0)
                i = tc.astype(jnp.int32)
                fr = tc - i.astype(jnp.float32)
                w0 = jnp.where(keep, 1.0 - fr, 0.0)
                w1 = jnp.where(keep, fr, 0.0)
                idx = (s * _HSIZE) + lane_base + i
                plsc.addupdate_scatter(hist, [idx], w0)
                plsc.addupdate_scatter(hist, [idx + 1], w1)
            return _

        lax.fori_loop(0, _CHUNK // (_LANES * _NBANK), vec_body, None,
                      unroll=2)
        return _

    lax.fori_loop(0, _NCHUNK, chunk_body, None)
    pltpu.sync_copy(hist, out_hbm.at[wid])


def _hist_sc_body(img0_hbm, img1_hbm, hmin_hbm, out0_hbm, out1_hbm,
                  buf, hist, hv):
    wid = lax.axis_index("s") * 2 + lax.axis_index("c")
    pltpu.sync_copy(hmin_hbm, hv)
    hminv = hv[...]
    dh = (0.0 - hminv) * (1.0 / (_NBIN - 1))
    invdh = 1.0 / dh
    _hist_worker(img0_hbm, out0_hbm, buf, hist, hminv, invdh, wid)
    _hist_worker(img1_hbm, out1_hbm, buf, hist, hminv, invdh, wid)


def _hist_sc(img0_flat, img1_flat, hmin_arr):
    mesh = plsc.VectorSubcoreMesh(core_axis_name="c", subcore_axis_name="s")
    f = functools.partial(
        pl.kernel,
        mesh=mesh,
        out_type=[
            jax.ShapeDtypeStruct((_NWORKERS, _HTOT), jnp.float32),
            jax.ShapeDtypeStruct((_NWORKERS, _HTOT), jnp.float32),
        ],
        scratch_types=[
            pltpu.VMEM((_CHUNK,), jnp.float32),
            pltpu.VMEM((_HTOT,), jnp.float32),
            pltpu.VMEM((_LANES,), jnp.float32),
        ],
        compiler_params=pltpu.CompilerParams(needs_layout_passes=False),
    )(_hist_sc_body)
    return f(img0_flat, img1_flat, hmin_arr)


def _kl_body(p0_ref, p1_ref, o_ref):
    eps = 1e-10
    h0 = jnp.sum(p0_ref[...], axis=0, keepdims=True)[:, :_NBIN]
    h1 = jnp.sum(p1_ref[...], axis=0, keepdims=True)[:, :_NBIN]
    h0 = (h0 + eps) / (jnp.sum(h0) + eps)
    h1 = (h1 + eps) / (jnp.sum(h1) + eps)
    inp = jnp.log((h1 + eps) / h1)
    tgt = jnp.log((h1 + eps) / h0)
    o_ref[0, 0] = jnp.mean(jnp.exp(tgt) * (tgt - inp))


def _kl(parts0, parts1):
    return pl.pallas_call(
        _kl_body,
        out_specs=pl.BlockSpec(memory_space=pltpu.SMEM),
        out_shape=jax.ShapeDtypeStruct((1, 1), jnp.float32),
    )(parts0, parts1)


def kernel(img0, img1):
    m = _global_min(img0)[0, 0]
    hmin_arr = jnp.full((_LANES,), m, dtype=jnp.float32)
    parts0, parts1 = _hist_sc(img0.reshape(-1), img1.reshape(-1), hmin_arr)
    p0 = parts0.reshape(_NWORKERS * _NBANK * _LANES, _HROW)
    p1 = parts1.reshape(_NWORKERS * _NBANK * _LANES, _HROW)
    return _kl(p0, p1)[0, 0]


# speedup vs baseline: 2.2633x; 2.2633x over previous
import functools

import jax
import jax.numpy as jnp
from jax import lax
from jax.experimental import pallas as pl
from jax.experimental.pallas import tpu as pltpu
from jax.experimental.pallas import tpu_sc as plsc

_NBIN = 256
_N = 4096 * 4096
_NWORKERS = 32
_EPW = _N // _NWORKERS
_CHUNK = 16384
_NCHUNK = _EPW // _CHUNK
_LANES = 16
_HROW = 264
_HSIZE = _LANES * _HROW
_NBANK = 4
_HTOT = _NBANK * _HSIZE


def _min_body(x_ref, o_ref):
    i = pl.program_id(0)
    m = jnp.min(x_ref[...])

    @pl.when(i == 0)
    def _():
        o_ref[0, 0] = m

    @pl.when(i > 0)
    def _():
        o_ref[0, 0] = jnp.minimum(o_ref[0, 0], m)


def _global_min(img):
    return pl.pallas_call(
        _min_body,
        grid=(16,),
        in_specs=[pl.BlockSpec((256, 4096), lambda i: (i, 0))],
        out_specs=pl.BlockSpec(memory_space=pltpu.SMEM),
        out_shape=jax.ShapeDtypeStruct((1, 1), jnp.float32),
    )(img)


def _hist_worker(img_hbm, out_hbm, buf, hist, hminv, invdh, wid):
    lane_base = jnp.arange(_LANES, dtype=jnp.int32) * _HROW
    zeros16 = jnp.zeros((_LANES,), jnp.float32)

    def zero_body(j, _):
        hist[pl.ds(j * _LANES, _LANES)] = zeros16
        return _

    lax.fori_loop(0, _HTOT // _LANES, zero_body, None)

    base = wid * _EPW

    def chunk_body(c, _):
        pltpu.sync_copy(img_hbm.at[pl.ds(base + c * _CHUNK, _CHUNK)], buf)

        @plsc.parallel_loop(0, _CHUNK // (_LANES * _NBANK), unroll=2)
        def vec_body(v):
            for s in range(_NBANK):
                x = buf[pl.ds((v * _NBANK + s) * _LANES, _LANES)]
                keep = jnp.logical_and(x >= hminv, x <= 0.0)
                t = (x - hminv) * invdh
                tc = jnp.clip(t, 0.0, 255.0)
                i = tc.astype(jnp.int32)
                fr = tc - i.astype(jnp.float32)
                w0 = jnp.where(keep, 1.0 - fr, 0.0)
                w1 = jnp.where(keep, fr, 0.0)
                idx = (s * _HSIZE) + lane_base + i
                plsc.addupdate_scatter(hist, [idx], w0)
                plsc.addupdate_scatter(hist, [idx + 1], w1)
        return _

    lax.fori_loop(0, _NCHUNK, chunk_body, None)
    pltpu.sync_copy(hist, out_hbm.at[wid])


def _hist_sc_body(img0_hbm, img1_hbm, hmin_hbm, out0_hbm, out1_hbm,
                  buf, hist, hv):
    wid = lax.axis_index("s") * 2 + lax.axis_index("c")
    pltpu.sync_copy(hmin_hbm, hv)
    hminv = hv[...]
    dh = (0.0 - hminv) * (1.0 / (_NBIN - 1))
    invdh = 1.0 / dh
    _hist_worker(img0_hbm, out0_hbm, buf, hist, hminv, invdh, wid)
    _hist_worker(img1_hbm, out1_hbm, buf, hist, hminv, invdh, wid)


def _hist_sc(img0_flat, img1_flat, hmin_arr):
    mesh = plsc.VectorSubcoreMesh(core_axis_name="c", subcore_axis_name="s")
    f = functools.partial(
        pl.kernel,
        mesh=mesh,
        out_type=[
            jax.ShapeDtypeStruct((_NWORKERS, _HTOT), jnp.float32),
            jax.ShapeDtypeStruct((_NWORKERS, _HTOT), jnp.float32),
        ],
        scratch_types=[
            pltpu.VMEM((_CHUNK,), jnp.float32),
            pltpu.VMEM((_HTOT,), jnp.float32),
            pltpu.VMEM((_LANES,), jnp.float32),
        ],
        compiler_params=pltpu.CompilerParams(needs_layout_passes=False),
    )(_hist_sc_body)
    return f(img0_flat, img1_flat, hmin_arr)


def _kl_body(p0_ref, p1_ref, o_ref):
    eps = 1e-10
    h0 = jnp.sum(p0_ref[...], axis=0, keepdims=True)[:, :_NBIN]
    h1 = jnp.sum(p1_ref[...], axis=0, keepdims=True)[:, :_NBIN]
    h0 = (h0 + eps) / (jnp.sum(h0) + eps)
    h1 = (h1 + eps) / (jnp.sum(h1) + eps)
    inp = jnp.log((h1 + eps) / h1)
    tgt = jnp.log((h1 + eps) / h0)
    o_ref[0, 0] = jnp.mean(jnp.exp(tgt) * (tgt - inp))


def _kl(parts0, parts1):
    return pl.pallas_call(
        _kl_body,
        out_specs=pl.BlockSpec(memory_space=pltpu.SMEM),
        out_shape=jax.ShapeDtypeStruct((1, 1), jnp.float32),
    )(parts0, parts1)


def kernel(img0, img1):
    m = _global_min(img0)[0, 0]
    hmin_arr = jnp.full((_LANES,), m, dtype=jnp.float32)
    parts0, parts1 = _hist_sc(img0.reshape(-1), img1.reshape(-1), hmin_arr)
    p0 = parts0.reshape(_NWORKERS * _NBANK * _LANES, _HROW)
    p1 = parts1.reshape(_NWORKERS * _NBANK * _LANES, _HROW)
    return _kl(p0, p1)[0, 0]


# speedup vs baseline: 2.9315x; 1.2952x over previous
import functools

import jax
import jax.numpy as jnp
from jax import lax
from jax.experimental import pallas as pl
from jax.experimental.pallas import tpu as pltpu
from jax.experimental.pallas import tpu_sc as plsc

_NBIN = 256
_N = 4096 * 4096
_NWORKERS = 32
_EPW = _N // _NWORKERS
_CHUNK = 16384
_NCHUNK = _EPW // _CHUNK
_LANES = 16
_HROW = 264
_HSIZE = _LANES * _HROW
_NBANK = 4
_HTOT = _NBANK * _HSIZE


def _min_body(x_ref, o_ref):
    i = pl.program_id(0)
    m = jnp.min(x_ref[...])

    @pl.when(i == 0)
    def _():
        o_ref[0, 0] = m

    @pl.when(i > 0)
    def _():
        o_ref[0, 0] = jnp.minimum(o_ref[0, 0], m)


def _global_min(img):
    return pl.pallas_call(
        _min_body,
        grid=(16,),
        in_specs=[pl.BlockSpec((256, 4096), lambda i: (i, 0))],
        out_specs=pl.BlockSpec(memory_space=pltpu.SMEM),
        out_shape=jax.ShapeDtypeStruct((1, 1), jnp.float32),
    )(img)


def _hist_worker(img_hbm, out_hbm, buf, hist, hminv, invdh, wid):
    lane_base = jnp.arange(_LANES, dtype=jnp.int32) * _HROW
    zeros16 = jnp.zeros((_LANES,), jnp.float32)

    def zero_body(j, _):
        hist[pl.ds(j * _LANES, _LANES)] = zeros16
        return _

    lax.fori_loop(0, _HTOT // _LANES, zero_body, None)

    base = wid * _EPW

    def chunk_body(c, _):
        pltpu.sync_copy(img_hbm.at[pl.ds(base + c * _CHUNK, _CHUNK)], buf)

        @plsc.parallel_loop(0, _CHUNK // (_LANES * _NBANK), unroll=4)
        def vec_body(v):
            for s in range(_NBANK):
                x = buf[pl.ds((v * _NBANK + s) * _LANES, _LANES)]
                keep = jnp.logical_and(x >= hminv, x <= 0.0)
                t = (x - hminv) * invdh
                tc = jnp.clip(t, 0.0, 255.0)
                i = tc.astype(jnp.int32)
                fr = tc - i.astype(jnp.float32)
                idx = (s * _HSIZE) + lane_base + i
                plsc.addupdate_scatter(hist, [idx], 1.0 - fr, mask=keep)
                plsc.addupdate_scatter(hist, [idx + 1], fr, mask=keep)
        return _

    lax.fori_loop(0, _NCHUNK, chunk_body, None)
    pltpu.sync_copy(hist, out_hbm.at[wid])


def _hist_sc_body(img0_hbm, img1_hbm, hmin_hbm, out0_hbm, out1_hbm,
                  buf, hist, hv):
    wid = lax.axis_index("s") * 2 + lax.axis_index("c")
    pltpu.sync_copy(hmin_hbm, hv)
    hminv = hv[...]
    dh = (0.0 - hminv) * (1.0 / (_NBIN - 1))
    invdh = 1.0 / dh
    _hist_worker(img0_hbm, out0_hbm, buf, hist, hminv, invdh, wid)
    _hist_worker(img1_hbm, out1_hbm, buf, hist, hminv, invdh, wid)


def _hist_sc(img0_flat, img1_flat, hmin_arr):
    mesh = plsc.VectorSubcoreMesh(core_axis_name="c", subcore_axis_name="s")
    f = functools.partial(
        pl.kernel,
        mesh=mesh,
        out_type=[
            jax.ShapeDtypeStruct((_NWORKERS, _HTOT), jnp.float32),
            jax.ShapeDtypeStruct((_NWORKERS, _HTOT), jnp.float32),
        ],
        scratch_types=[
            pltpu.VMEM((_CHUNK,), jnp.float32),
            pltpu.VMEM((_HTOT,), jnp.float32),
            pltpu.VMEM((_LANES,), jnp.float32),
        ],
        compiler_params=pltpu.CompilerParams(needs_layout_passes=False),
    )(_hist_sc_body)
    return f(img0_flat, img1_flat, hmin_arr)


def _kl_body(p0_ref, p1_ref, o_ref):
    eps = 1e-10
    h0 = jnp.sum(p0_ref[...], axis=0, keepdims=True)[:, :_NBIN]
    h1 = jnp.sum(p1_ref[...], axis=0, keepdims=True)[:, :_NBIN]
    h0 = (h0 + eps) / (jnp.sum(h0) + eps)
    h1 = (h1 + eps) / (jnp.sum(h1) + eps)
    inp = jnp.log((h1 + eps) / h1)
    tgt = jnp.log((h1 + eps) / h0)
    o_ref[0, 0] = jnp.mean(jnp.exp(tgt) * (tgt - inp))


def _kl(parts0, parts1):
    return pl.pallas_call(
        _kl_body,
        out_specs=pl.BlockSpec(memory_space=pltpu.SMEM),
        out_shape=jax.ShapeDtypeStruct((1, 1), jnp.float32),
    )(parts0, parts1)


def kernel(img0, img1):
    m = _global_min(img0)[0, 0]
    hmin_arr = jnp.full((_LANES,), m, dtype=jnp.float32)
    parts0, parts1 = _hist_sc(img0.reshape(-1), img1.reshape(-1), hmin_arr)
    p0 = parts0.reshape(_NWORKERS * _NBANK * _LANES, _HROW)
    p1 = parts1.reshape(_NWORKERS * _NBANK * _LANES, _HROW)
    return _kl(p0, p1)[0, 0]


# speedup vs baseline: 2.9446x; 1.0044x over previous
import functools

import jax
import jax.numpy as jnp
from jax import lax
from jax.experimental import pallas as pl
from jax.experimental.pallas import tpu as pltpu
from jax.experimental.pallas import tpu_sc as plsc

_NBIN = 256
_N = 4096 * 4096
_NWORKERS = 32
_EPW = _N // _NWORKERS
_CHUNK = 16384
_NCHUNK = _EPW // _CHUNK
_LANES = 16
_HROW = 264
_HSIZE = _LANES * _HROW
_NBANK = 4
_HTOT = _NBANK * _HSIZE


def _min_body(x_ref, o_ref):
    i = pl.program_id(0)
    m = jnp.min(x_ref[...])

    @pl.when(i == 0)
    def _():
        o_ref[0, 0] = m

    @pl.when(i > 0)
    def _():
        o_ref[0, 0] = jnp.minimum(o_ref[0, 0], m)


def _global_min(img):
    return pl.pallas_call(
        _min_body,
        grid=(16,),
        in_specs=[pl.BlockSpec((256, 4096), lambda i: (i, 0))],
        out_specs=pl.BlockSpec(memory_space=pltpu.SMEM),
        out_shape=jax.ShapeDtypeStruct((1, 1), jnp.float32),
    )(img)


def _hist_worker(img_hbm, out_hbm, buf, hist, hminv, invdh, wid):
    lane_base = jnp.arange(_LANES, dtype=jnp.int32) * _HROW
    zeros16 = jnp.zeros((_LANES,), jnp.float32)

    def zero_body(j, _):
        hist[pl.ds(j * _LANES, _LANES)] = zeros16
        return _

    lax.fori_loop(0, _HTOT // _LANES, zero_body, None)

    base = wid * _EPW

    def chunk_body(c, _):
        pltpu.sync_copy(img_hbm.at[pl.ds(base + c * _CHUNK, _CHUNK)], buf)

        @plsc.parallel_loop(0, _CHUNK // (_LANES * _NBANK), unroll=8)
        def vec_body(v):
            for s in range(_NBANK):
                x = buf[pl.ds((v * _NBANK + s) * _LANES, _LANES)]
                keep = jnp.logical_and(x >= hminv, x <= 0.0)
                t = (x - hminv) * invdh
                tc = jnp.clip(t, 0.0, 255.0)
                i = tc.astype(jnp.int32)
                fr = tc - i.astype(jnp.float32)
                idx = (s * _HSIZE) + lane_base + i
                plsc.addupdate_scatter(hist, [idx], 1.0 - fr, mask=keep)
                plsc.addupdate_scatter(hist, [idx + 1], fr, mask=keep)
        return _

    lax.fori_loop(0, _NCHUNK, chunk_body, None)
    pltpu.sync_copy(hist, out_hbm.at[wid])


def _hist_sc_body(img0_hbm, img1_hbm, hmin_hbm, out0_hbm, out1_hbm,
                  buf, hist, hv):
    wid = lax.axis_index("s") * 2 + lax.axis_index("c")
    pltpu.sync_copy(hmin_hbm, hv)
    hminv = hv[...]
    dh = (0.0 - hminv) * (1.0 / (_NBIN - 1))
    invdh = 1.0 / dh
    _hist_worker(img0_hbm, out0_hbm, buf, hist, hminv, invdh, wid)
    _hist_worker(img1_hbm, out1_hbm, buf, hist, hminv, invdh, wid)


def _hist_sc(img0_flat, img1_flat, hmin_arr):
    mesh = plsc.VectorSubcoreMesh(core_axis_name="c", subcore_axis_name="s")
    f = functools.partial(
        pl.kernel,
        mesh=mesh,
        out_type=[
            jax.ShapeDtypeStruct((_NWORKERS, _HTOT), jnp.float32),
            jax.ShapeDtypeStruct((_NWORKERS, _HTOT), jnp.float32),
        ],
        scratch_types=[
            pltpu.VMEM((_CHUNK,), jnp.float32),
            pltpu.VMEM((_HTOT,), jnp.float32),
            pltpu.VMEM((_LANES,), jnp.float32),
        ],
        compiler_params=pltpu.CompilerParams(needs_layout_passes=False),
    )(_hist_sc_body)
    return f(img0_flat, img1_flat, hmin_arr)


def _kl_body(p0_ref, p1_ref, o_ref):
    eps = 1e-10
    h0 = jnp.sum(p0_ref[...], axis=0, keepdims=True)[:, :_NBIN]
    h1 = jnp.sum(p1_ref[...], axis=0, keepdims=True)[:, :_NBIN]
    h0 = (h0 + eps) / (jnp.sum(h0) + eps)
    h1 = (h1 + eps) / (jnp.sum(h1) + eps)
    inp = jnp.log((h1 + eps) / h1)
    tgt = jnp.log((h1 + eps) / h0)
    o_ref[0, 0] = jnp.mean(jnp.exp(tgt) * (tgt - inp))


def _kl(parts0, parts1):
    return pl.pallas_call(
        _kl_body,
        out_specs=pl.BlockSpec(memory_space=pltpu.SMEM),
        out_shape=jax.ShapeDtypeStruct((1, 1), jnp.float32),
    )(parts0, parts1)


def kernel(img0, img1):
    m = _global_min(img0)[0, 0]
    hmin_arr = jnp.full((_LANES,), m, dtype=jnp.float32)
    parts0, parts1 = _hist_sc(img0.reshape(-1), img1.reshape(-1), hmin_arr)
    p0 = parts0.reshape(_NWORKERS * _NBANK * _LANES, _HROW)
    p1 = parts1.reshape(_NWORKERS * _NBANK * _LANES, _HROW)
    return _kl(p0, p1)[0, 0]


# speedup vs baseline: 3.5125x; 1.1929x over previous
import functools

import jax
import jax.numpy as jnp
from jax import lax
from jax.experimental import pallas as pl
from jax.experimental.pallas import tpu as pltpu
from jax.experimental.pallas import tpu_sc as plsc

_NBIN = 256
_N = 4096 * 4096
_NWORKERS = 32
_EPW = _N // _NWORKERS
_CHUNK = 16384
_NCHUNK = _EPW // _CHUNK
_LANES = 16
_HROW = 264
_HSIZE = _LANES * _HROW
_NBANK = 4
_HTOT = _NBANK * _HSIZE


def _min_body(x_ref, o_ref):
    i = pl.program_id(0)
    m = jnp.min(x_ref[...])

    @pl.when(i == 0)
    def _():
        o_ref[0, 0] = m

    @pl.when(i > 0)
    def _():
        o_ref[0, 0] = jnp.minimum(o_ref[0, 0], m)


def _global_min(img):
    return pl.pallas_call(
        _min_body,
        grid=(16,),
        in_specs=[pl.BlockSpec((256, 4096), lambda i: (i, 0))],
        out_specs=pl.BlockSpec(memory_space=pltpu.SMEM),
        out_shape=jax.ShapeDtypeStruct((1, 1), jnp.float32),
    )(img)


def _hist_worker(img_hbm, out_hbm, bufs, sems, hist, hminv, invdh, wid):
    lane_base = jnp.arange(_LANES, dtype=jnp.int32) * _HROW
    zeros16 = jnp.zeros((_LANES,), jnp.float32)
    base = wid * _EPW

    def start(j, b):
        pltpu.make_async_copy(
            img_hbm.at[pl.ds(base + j * _CHUNK, _CHUNK)], bufs[b], sems[b]
        ).start()

    def wait(b):
        pltpu.make_async_copy(
            img_hbm.at[pl.ds(base, _CHUNK)], bufs[b], sems[b]
        ).wait()

    start(0, 0)

    def zero_body(j, _):
        hist[pl.ds(j * _LANES, _LANES)] = zeros16
        return _

    lax.fori_loop(0, _HTOT // _LANES, zero_body, None)

    def process(buf):
        @plsc.parallel_loop(0, _CHUNK // (_LANES * _NBANK), unroll=8)
        def vec_body(v):
            for s in range(_NBANK):
                x = buf[pl.ds((v * _NBANK + s) * _LANES, _LANES)]
                keep = jnp.logical_and(x >= hminv, x <= 0.0)
                t = (x - hminv) * invdh
                tc = jnp.clip(t, 0.0, 255.0)
                i = tc.astype(jnp.int32)
                fr = tc - i.astype(jnp.float32)
                idx = (s * _HSIZE) + lane_base + i
                plsc.addupdate_scatter(hist, [idx], 1.0 - fr, mask=keep)
                plsc.addupdate_scatter(hist, [idx + 1], fr, mask=keep)

    def chunk_body(k, _):
        for b in range(2):
            j = 2 * k + b

            @pl.when(j + 1 < _NCHUNK)
            def _():
                start(j + 1, 1 - b)

            wait(b)
            process(bufs[b])
        return _

    lax.fori_loop(0, _NCHUNK // 2, chunk_body, None)
    pltpu.sync_copy(hist, out_hbm.at[wid])


def _hist_sc_body(img0_hbm, img1_hbm, hmin_hbm, out0_hbm, out1_hbm,
                  buf0, buf1, hist, hv, sem0, sem1):
    wid = lax.axis_index("s") * 2 + lax.axis_index("c")
    pltpu.sync_copy(hmin_hbm, hv)
    hminv = hv[...]
    dh = (0.0 - hminv) * (1.0 / (_NBIN - 1))
    invdh = 1.0 / dh
    bufs = (buf0, buf1)
    sems = (sem0, sem1)
    _hist_worker(img0_hbm, out0_hbm, bufs, sems, hist, hminv, invdh, wid)
    _hist_worker(img1_hbm, out1_hbm, bufs, sems, hist, hminv, invdh, wid)


def _hist_sc(img0_flat, img1_flat, hmin_arr):
    mesh = plsc.VectorSubcoreMesh(core_axis_name="c", subcore_axis_name="s")
    f = functools.partial(
        pl.kernel,
        mesh=mesh,
        out_type=[
            jax.ShapeDtypeStruct((_NWORKERS, _HTOT), jnp.float32),
            jax.ShapeDtypeStruct((_NWORKERS, _HTOT), jnp.float32),
        ],
        scratch_types=[
            pltpu.VMEM((_CHUNK,), jnp.float32),
            pltpu.VMEM((_CHUNK,), jnp.float32),
            pltpu.VMEM((_HTOT,), jnp.float32),
            pltpu.VMEM((_LANES,), jnp.float32),
            pltpu.SemaphoreType.DMA,
            pltpu.SemaphoreType.DMA,
        ],
        compiler_params=pltpu.CompilerParams(needs_layout_passes=False),
    )(_hist_sc_body)
    return f(img0_flat, img1_flat, hmin_arr)


def _kl_body(p0_ref, p1_ref, o_ref):
    eps = 1e-10
    h0 = jnp.sum(p0_ref[...], axis=0, keepdims=True)[:, :_NBIN]
    h1 = jnp.sum(p1_ref[...], axis=0, keepdims=True)[:, :_NBIN]
    h0 = (h0 + eps) / (jnp.sum(h0) + eps)
    h1 = (h1 + eps) / (jnp.sum(h1) + eps)
    inp = jnp.log((h1 + eps) / h1)
    tgt = jnp.log((h1 + eps) / h0)
    o_ref[0, 0] = jnp.mean(jnp.exp(tgt) * (tgt - inp))


def _kl(parts0, parts1):
    return pl.pallas_call(
        _kl_body,
        out_specs=pl.BlockSpec(memory_space=pltpu.SMEM),
        out_shape=jax.ShapeDtypeStruct((1, 1), jnp.float32),
    )(parts0, parts1)


def kernel(img0, img1):
    m = _global_min(img0)[0, 0]
    hmin_arr = jnp.full((_LANES,), m, dtype=jnp.float32)
    parts0, parts1 = _hist_sc(img0.reshape(-1), img1.reshape(-1), hmin_arr)
    p0 = parts0.reshape(_NWORKERS * _NBANK * _LANES, _HROW)
    p1 = parts1.reshape(_NWORKERS * _NBANK * _LANES, _HROW)
    return _kl(p0, p1)[0, 0]


# speedup vs baseline: 4.5024x; 1.2818x over previous
import functools

import jax
import jax.numpy as jnp
from jax import lax
from jax.experimental import pallas as pl
from jax.experimental.pallas import tpu as pltpu
from jax.experimental.pallas import tpu_sc as plsc

_NBIN = 256
_NROW = 4096
_NCOL = 4096
_NWORKERS = 32
_ROWS_PW = _NROW // _NWORKERS
_SLAB = 8
_NCHUNK = _ROWS_PW // _SLAB
_VECS = _SLAB * _NCOL // 16
_LANES = 16
_HROW = 264
_HSIZE = _LANES * _HROW
_NBANK = 4
_HTOT = _NBANK * _HSIZE


def _min_body(x_ref, o_ref):
    i = pl.program_id(0)
    m = jnp.min(x_ref[...])

    @pl.when(i == 0)
    def _():
        o_ref[0, 0] = m

    @pl.when(i > 0)
    def _():
        o_ref[0, 0] = jnp.minimum(o_ref[0, 0], m)


def _global_min(img):
    return pl.pallas_call(
        _min_body,
        grid=(16,),
        in_specs=[pl.BlockSpec((256, 4096), lambda i: (i, 0))],
        out_specs=pl.BlockSpec(memory_space=pltpu.SMEM),
        out_shape=jax.ShapeDtypeStruct((1, 1), jnp.float32),
    )(img)


def _hist_worker(img_hbm, out_hbm, bufs, sems, hist, hminv, invdh, wid):
    lane_base = jnp.arange(_LANES, dtype=jnp.int32) * _HROW
    zeros16 = jnp.zeros((_LANES,), jnp.float32)
    row0 = wid * _ROWS_PW

    def start(j, b):
        pltpu.make_async_copy(
            img_hbm.at[pl.ds(row0 + j * _SLAB, _SLAB), :], bufs[b], sems[b]
        ).start()

    def wait(b):
        pltpu.make_async_copy(
            img_hbm.at[pl.ds(row0, _SLAB), :], bufs[b], sems[b]
        ).wait()

    start(0, 0)

    def zero_body(j, _):
        hist[pl.ds(j * _LANES, _LANES)] = zeros16
        return _

    lax.fori_loop(0, _HTOT // _LANES, zero_body, None)

    def process(buf):
        @plsc.parallel_loop(0, _VECS // _NBANK, unroll=8)
        def vec_body(v):
            for s in range(_NBANK):
                ln = v * _NBANK + s
                r = ln // (_NCOL // _LANES)
                c = (ln % (_NCOL // _LANES)) * _LANES
                x = buf[r, pl.ds(c, _LANES)]
                keep = jnp.logical_and(x >= hminv, x <= 0.0)
                t = (x - hminv) * invdh
                tc = jnp.clip(t, 0.0, 255.0)
                i = tc.astype(jnp.int32)
                fr = tc - i.astype(jnp.float32)
                idx = (s * _HSIZE) + lane_base + i
                plsc.addupdate_scatter(hist, [idx], 1.0 - fr, mask=keep)
                plsc.addupdate_scatter(hist, [idx + 1], fr, mask=keep)

    def chunk_body(k, _):
        for b in range(2):
            j = 2 * k + b

            @pl.when(j + 1 < _NCHUNK)
            def _():
                start(j + 1, 1 - b)

            wait(b)
            process(bufs[b])
        return _

    lax.fori_loop(0, _NCHUNK // 2, chunk_body, None)
    pltpu.sync_copy(hist, out_hbm.at[wid])


def _hist_sc_body(img0_hbm, img1_hbm, hmin_hbm, out0_hbm, out1_hbm,
                  buf0, buf1, hist, hv, sem0, sem1):
    wid = lax.axis_index("s") * 2 + lax.axis_index("c")
    pltpu.sync_copy(hmin_hbm, hv)
    hminv = hv[...]
    dh = (0.0 - hminv) * (1.0 / (_NBIN - 1))
    invdh = 1.0 / dh
    bufs = (buf0, buf1)
    sems = (sem0, sem1)
    _hist_worker(img0_hbm, out0_hbm, bufs, sems, hist, hminv, invdh, wid)
    _hist_worker(img1_hbm, out1_hbm, bufs, sems, hist, hminv, invdh, wid)


def _hist_sc(img0_flat, img1_flat, hmin_arr):
    mesh = plsc.VectorSubcoreMesh(core_axis_name="c", subcore_axis_name="s")
    f = functools.partial(
        pl.kernel,
        mesh=mesh,
        out_type=[
            jax.ShapeDtypeStruct((_NWORKERS, _HTOT), jnp.float32),
            jax.ShapeDtypeStruct((_NWORKERS, _HTOT), jnp.float32),
        ],
        scratch_types=[
            pltpu.VMEM((_SLAB, _NCOL), jnp.float32),
            pltpu.VMEM((_SLAB, _NCOL), jnp.float32),
            pltpu.VMEM((_HTOT,), jnp.float32),
            pltpu.VMEM((_LANES,), jnp.float32),
            pltpu.SemaphoreType.DMA,
            pltpu.SemaphoreType.DMA,
        ],
        compiler_params=pltpu.CompilerParams(needs_layout_passes=False),
    )(_hist_sc_body)
    return f(img0_flat, img1_flat, hmin_arr)


def _kl_body(p0_ref, p1_ref, o_ref):
    eps = 1e-10
    h0 = jnp.sum(p0_ref[...], axis=0, keepdims=True)[:, :_NBIN]
    h1 = jnp.sum(p1_ref[...], axis=0, keepdims=True)[:, :_NBIN]
    h0 = (h0 + eps) / (jnp.sum(h0) + eps)
    h1 = (h1 + eps) / (jnp.sum(h1) + eps)
    inp = jnp.log((h1 + eps) / h1)
    tgt = jnp.log((h1 + eps) / h0)
    o_ref[0, 0] = jnp.mean(jnp.exp(tgt) * (tgt - inp))


def _kl(parts0, parts1):
    return pl.pallas_call(
        _kl_body,
        out_specs=pl.BlockSpec(memory_space=pltpu.SMEM),
        out_shape=jax.ShapeDtypeStruct((1, 1), jnp.float32),
    )(parts0, parts1)


def kernel(img0, img1):
    m = _global_min(img0)[0, 0]
    hmin_arr = jnp.full((_LANES,), m, dtype=jnp.float32)
    parts0, parts1 = _hist_sc(img0, img1, hmin_arr)
    p0 = parts0.reshape(_NWORKERS * _NBANK * _LANES, _HROW)
    p1 = parts1.reshape(_NWORKERS * _NBANK * _LANES, _HROW)
    return _kl(p0, p1)[0, 0]


# speedup vs baseline: 4.6285x; 1.0280x over previous
import functools

import jax
import jax.numpy as jnp
from jax import lax
from jax.experimental import pallas as pl
from jax.experimental.pallas import tpu as pltpu
from jax.experimental.pallas import tpu_sc as plsc

_NBIN = 256
_NROW = 4096
_NCOL = 4096
_NWORKERS = 32
_ROWS_PW = _NROW // _NWORKERS
_SLAB = 8
_NCHUNK = _ROWS_PW // _SLAB
_VECS = _SLAB * _NCOL // 16
_LANES = 16
_HROW = 264
_HSIZE = _LANES * _HROW
_NBANK = 4
_HTOT = _NBANK * _HSIZE


def _min_body(x_ref, o_ref):
    i = pl.program_id(0)
    m = jnp.min(x_ref[...])

    @pl.when(i == 0)
    def _():
        o_ref[0, 0] = m

    @pl.when(i > 0)
    def _():
        o_ref[0, 0] = jnp.minimum(o_ref[0, 0], m)


def _global_min(img):
    return pl.pallas_call(
        _min_body,
        grid=(16,),
        in_specs=[pl.BlockSpec((256, 4096), lambda i: (i, 0))],
        out_specs=pl.BlockSpec(memory_space=pltpu.SMEM),
        out_shape=jax.ShapeDtypeStruct((1, 1), jnp.float32),
    )(img)


def _hist_worker(img_hbm, out_hbm, bufs, sems, hist, hminv, invdh, wid):
    lane_base = jnp.arange(_LANES, dtype=jnp.int32) * _HROW
    zeros16 = jnp.zeros((_LANES,), jnp.float32)
    row0 = wid * _ROWS_PW

    def start(j, b):
        pltpu.make_async_copy(
            img_hbm.at[pl.ds(row0 + j * _SLAB, _SLAB), :], bufs[b], sems[b]
        ).start()

    def wait(b):
        pltpu.make_async_copy(
            img_hbm.at[pl.ds(row0, _SLAB), :], bufs[b], sems[b]
        ).wait()

    start(0, 0)

    def zero_body(j, _):
        hist[pl.ds(j * _LANES, _LANES)] = zeros16
        return _

    lax.fori_loop(0, _HTOT // _LANES, zero_body, None)

    def process(buf):
        @plsc.parallel_loop(0, _VECS // _NBANK, unroll=8)
        def vec_body(v):
            for s in range(_NBANK):
                ln = v * _NBANK + s
                r = ln // (_NCOL // _LANES)
                c = (ln % (_NCOL // _LANES)) * _LANES
                x = buf[r, pl.ds(c, _LANES)]
                keep = jnp.logical_and(x >= hminv, x <= 0.0)
                t = (x - hminv) * invdh
                tc = jnp.clip(t, 0.0, 255.0)
                i = tc.astype(jnp.int32)
                fr = tc - i.astype(jnp.float32)
                idx = (s * _HSIZE) + lane_base + i
                plsc.addupdate_scatter(hist, [idx], 1.0 - fr, mask=keep)
                plsc.addupdate_scatter(hist, [idx + 1], fr, mask=keep)

    def chunk_body(k, _):
        for b in range(2):
            j = 2 * k + b

            @pl.when(j + 1 < _NCHUNK)
            def _():
                start(j + 1, 1 - b)

            wait(b)
            process(bufs[b])
        return _

    lax.fori_loop(0, _NCHUNK // 2, chunk_body, None)
    pltpu.sync_copy(hist, out_hbm.at[wid])


def _hist_sc_body(img0_hbm, img1_hbm, hmin_hbm, out0_hbm, out1_hbm,
                  buf0, buf1, hist, hv, sem0, sem1):
    wid = lax.axis_index("s") * 2 + lax.axis_index("c")
    pltpu.sync_copy(hmin_hbm, hv)
    hminv = hv[...]
    dh = (0.0 - hminv) * (1.0 / (_NBIN - 1))
    invdh = 1.0 / dh
    bufs = (buf0, buf1)
    sems = (sem0, sem1)
    _hist_worker(img0_hbm, out0_hbm, bufs, sems, hist, hminv, invdh, wid)
    _hist_worker(img1_hbm, out1_hbm, bufs, sems, hist, hminv, invdh, wid)


def _hist_sc(img0_flat, img1_flat, hmin_arr):
    mesh = plsc.VectorSubcoreMesh(core_axis_name="c", subcore_axis_name="s")
    f = functools.partial(
        pl.kernel,
        mesh=mesh,
        out_type=[
            jax.ShapeDtypeStruct((_NWORKERS, _HTOT), jnp.float32),
            jax.ShapeDtypeStruct((_NWORKERS, _HTOT), jnp.float32),
        ],
        scratch_types=[
            pltpu.VMEM((_SLAB, _NCOL), jnp.float32),
            pltpu.VMEM((_SLAB, _NCOL), jnp.float32),
            pltpu.VMEM((_HTOT,), jnp.float32),
            pltpu.VMEM((_LANES,), jnp.float32),
            pltpu.SemaphoreType.DMA,
            pltpu.SemaphoreType.DMA,
        ],
        compiler_params=pltpu.CompilerParams(needs_layout_passes=False),
    )(_hist_sc_body)
    return f(img0_flat, img1_flat, hmin_arr)


def _kl_body(p0_ref, p1_ref, o_ref):
    eps = 1e-10

    def _merge(ref):
        acc = ref[:, 0:_HROW]
        for g in range(1, _HTOT // _HROW):
            acc = acc + ref[:, g * _HROW:(g + 1) * _HROW]
        return jnp.sum(acc, axis=0, keepdims=True)[:, :_NBIN]

    h0 = _merge(p0_ref)
    h1 = _merge(p1_ref)
    h0 = (h0 + eps) / (jnp.sum(h0) + eps)
    h1 = (h1 + eps) / (jnp.sum(h1) + eps)
    inp = jnp.log((h1 + eps) / h1)
    tgt = jnp.log((h1 + eps) / h0)
    o_ref[0, 0] = jnp.mean(jnp.exp(tgt) * (tgt - inp))


def _kl(parts0, parts1):
    return pl.pallas_call(
        _kl_body,
        out_specs=pl.BlockSpec(memory_space=pltpu.SMEM),
        out_shape=jax.ShapeDtypeStruct((1, 1), jnp.float32),
    )(parts0, parts1)


def kernel(img0, img1):
    m = _global_min(img0)[0, 0]
    hmin_arr = jnp.full((_LANES,), m, dtype=jnp.float32)
    parts0, parts1 = _hist_sc(img0, img1, hmin_arr)
    return _kl(parts0, parts1)[0, 0]
